# baseline (speedup 1.0000x reference)
import functools

import jax
import jax.numpy as jnp
from jax import lax
from jax.experimental import pallas as pl
from jax.experimental.pallas import tpu as pltpu
from jax.experimental.pallas import tpu_sc as plsc

K = 8
Q = 1024
D = 128
N = 100000
B = 16
QB = 1024
BLK = 1024
NSTEPS = -(-N // BLK)
NQB = Q // QB
C = NSTEPS * K

NC = 2
NS = 16
NW = NC * NS
QPW = Q // NW
IPW = QPW * K

INF = jnp.inf


def _cand_body(q_ref, x_ref, cv_out, ci_out, ubv, work_ref, fold_v, fold_i,
               fold2_v, fold2_i):
    j = pl.program_id(0)
    qb = pl.program_id(1)

    qv = q_ref[pl.ds(qb * QB, QB), :]
    xv = x_ref[...]
    qsq = jnp.sum(qv * qv, axis=1, keepdims=True)
    xsq = jnp.sum(xv * xv, axis=1)
    s = lax.dot_general(qv, xv, (((1,), (1,)), ((), ())),
                        preferred_element_type=jnp.float32)
    d2 = jnp.maximum(qsq + xsq[None, :] - 2.0 * s, 0.0)

    colsf = (jax.lax.broadcasted_iota(jnp.int32, (QB, BLK), 1)
             .astype(jnp.float32) + (j * BLK).astype(jnp.float32))

    qs = pl.ds(qb * QB, QB)

    @pl.when(j == 0)
    def _init():
        ubv[qs, :] = jnp.full((QB, K), INF, jnp.float32)

    thr = ubv[qs, K - 1:K]
    maskf = jnp.logical_and(d2 < thr, colsf < float(N)).astype(jnp.float32)
    cv_out[0, qs, :] = jnp.full((QB, K), INF, jnp.float32)
    ci_out[0, qs, :] = jnp.full((QB, K), INF, jnp.float32)

    P = jnp.where(maskf > 0.0, d2, INF)

    M = P[:, 0:128]
    Mi = jnp.zeros((QB, 128), jnp.float32)
    cnts = maskf[:, 0:128]
    for c in range(1, BLK // 128):
        v = P[:, c * 128:(c + 1) * 128]
        Mi = jnp.where(v < M, float(c), Mi)
        M = jnp.minimum(M, v)
        cnts = cnts + maskf[:, c * 128:(c + 1) * 128]
    cnt = jnp.max(jnp.sum(cnts, axis=1))
    ok = jnp.max(cnts) <= 1.0
    lanef = (jax.lax.broadcasted_iota(jnp.int32, (QB, 128), 1)
             .astype(jnp.float32))
    colglob = Mi * 128.0 + lanef + (j * BLK).astype(jnp.float32)

    @pl.when(ok)
    def _cheap():
        fold_v[...] = M
        fold_i[...] = colglob
        for k in range(K):
            @pl.when(cnt > float(k))
            def _p():
                w = fold_v[...]
                cg = fold_i[...]
                m = jnp.min(w, axis=1, keepdims=True)
                am = jnp.min(jnp.where(w == m, cg, INF), axis=1,
                             keepdims=True)
                cv_out[0, qs, K - 1 - k:K - k] = m
                ci_out[0, qs, K - 1 - k:K - k] = am
                if k < K - 1:
                    fold_v[...] = jnp.where(cg == am, INF, w)

    ok2 = jnp.max(cnts) <= 2.0

    @pl.when(jnp.logical_not(ok) & ok2)
    def _two():
        lo = P[:, 0:128]
        loi = jnp.zeros((QB, 128), jnp.float32)
        hi = jnp.full((QB, 128), INF, jnp.float32)
        hii = jnp.zeros((QB, 128), jnp.float32)
        for c in range(1, BLK // 128):
            t = P[:, c * 128:(c + 1) * 128]
            c1 = t < lo
            hii = jnp.where(c1, loi, jnp.where(t < hi, float(c), hii))
            hi = jnp.where(c1, lo, jnp.minimum(hi, t))
            loi = jnp.where(c1, float(c), loi)
            lo = jnp.minimum(lo, t)
        base = lanef + (j * BLK).astype(jnp.float32)
        fold2_v[:, 0:128] = lo
        fold2_v[:, 128:256] = hi
        fold2_i[:, 0:128] = loi * 128.0 + base
        fold2_i[:, 128:256] = hii * 128.0 + base
        for k in range(K):
            @pl.when(cnt > float(k))
            def _p2():
                w = fold2_v[...]
                cg = fold2_i[...]
                m = jnp.min(w, axis=1, keepdims=True)
                am = jnp.min(jnp.where(w == m, cg, INF), axis=1,
                             keepdims=True)
                cv_out[0, qs, K - 1 - k:K - k] = m
                ci_out[0, qs, K - 1 - k:K - k] = am
                if k < K - 1:
                    fold2_v[...] = jnp.where(cg == am, INF, w)

    @pl.when(jnp.logical_not(ok2))
    def _full():
        work_ref[...] = P
        for k in range(K):
            @pl.when(cnt > float(k))
            def _pass():
                work = work_ref[...]
                m = jnp.min(work, axis=1, keepdims=True)
                am = jnp.min(jnp.where(work == m, colsf, INF), axis=1,
                             keepdims=True)
                cv_out[0, qs, K - 1 - k:K - k] = m
                ci_out[0, qs, K - 1 - k:K - k] = am
                if k < K - 1:
                    work_ref[...] = jnp.where(colsf == am, INF, work)

    a = ubv[qs, :]
    b = cv_out[0, qs, :]
    L = jnp.minimum(a, b)
    lane = jax.lax.broadcasted_iota(jnp.int32, (QB, K), 1)
    for d in (4, 2, 1):
        fwd = jnp.concatenate(
            [L[:, d:], jnp.full((QB, d), INF, jnp.float32)], axis=1)
        bwd = jnp.concatenate(
            [jnp.full((QB, d), -INF, jnp.float32), L[:, :K - d]], axis=1)
        up = (lane // d) % 2 == 0
        L = jnp.where(up, jnp.minimum(L, fwd), jnp.maximum(L, bwd))
    ubv[qs, :] = L


def _candidates(x, q):
    return pl.pallas_call(
        _cand_body,
        grid=(NSTEPS, NQB),
        in_specs=[
            pl.BlockSpec((Q, D), lambda j, qb: (0, 0)),
            pl.BlockSpec((BLK, D), lambda j, qb: (j, 0)),
        ],
        out_specs=[
            pl.BlockSpec((1, Q, K), lambda j, qb: (j, 0, 0)),
            pl.BlockSpec((1, Q, K), lambda j, qb: (j, 0, 0)),
        ],
        out_shape=[
            jax.ShapeDtypeStruct((NSTEPS, Q, K), jnp.float32),
            jax.ShapeDtypeStruct((NSTEPS, Q, K), jnp.float32),
        ],
        scratch_shapes=[
            pltpu.VMEM((Q, K), jnp.float32),
            pltpu.VMEM((QB, BLK), jnp.float32),
            pltpu.VMEM((QB, 128), jnp.float32),
            pltpu.VMEM((QB, 128), jnp.float32),
            pltpu.VMEM((QB, 256), jnp.float32),
            pltpu.VMEM((QB, 256), jnp.float32),
        ],
    )(q, x)


def _final_body(cv_ref, ci_ref, idx_out, w_out):
    cw = cv_ref[...]
    ci = ci_ref[...]
    fv, fi = [], []
    for _ in range(K):
        m = jnp.min(cw, axis=1, keepdims=True)
        am = jnp.min(jnp.where(cw == m, ci, INF), axis=1, keepdims=True)
        fv.append(m)
        fi.append(am)
        cw = jnp.where(ci == am, INF, cw)
    d2t = jnp.concatenate(fv, axis=1)
    idx_out[...] = jnp.concatenate(fi, axis=1).astype(jnp.int32)
    dist = jnp.sqrt(d2t)
    sigma2 = jnp.square(jnp.max(dist)) / 9.0
    wgt = jnp.exp(-jnp.square(dist) / (2.0 * sigma2))
    w_out[...] = wgt / jnp.sum(wgt, axis=-1, keepdims=True)


def _topk_weights(cv, ci):
    return pl.pallas_call(
        _final_body,
        out_shape=[
            jax.ShapeDtypeStruct((Q, K), jnp.int32),
            jax.ShapeDtypeStruct((Q, K), jnp.float32),
        ],
    )(cv, ci)


def _sc_body(zt_hbm, idx_hbm, w_hbm, out_hbm, idx_v, w_v, rows_v, acc_v, sem):
    c = lax.axis_index("c")
    s = lax.axis_index("s")
    wid = s * NC + c
    base = wid * IPW
    qbase = wid * QPW
    pltpu.sync_copy(idx_hbm.at[pl.ds(base, IPW)], idx_v)
    pltpu.sync_copy(w_hbm.at[pl.ds(base, IPW)], w_v)
    pltpu.async_copy(zt_hbm.at[idx_v], rows_v, sem).wait()
    for i in range(QPW):
        acc = rows_v[i * K, :] * w_v[i * K, :]
        for k in range(1, K):
            acc = acc + rows_v[i * K + k, :] * w_v[i * K + k, :]
        acc_v[i, :] = acc
    pltpu.sync_copy(acc_v, out_hbm.at[pl.ds(qbase, QPW)])


def _sc_gather_sum(zt, idxf, wexp):
    run = pl.kernel(
        _sc_body,
        out_type=jax.ShapeDtypeStruct((Q, B), jnp.float32),
        mesh=plsc.VectorSubcoreMesh(core_axis_name="c", subcore_axis_name="s",
                                    num_cores=NC, num_subcores=NS),
        scratch_types=[
            pltpu.VMEM((IPW,), jnp.int32),
            pltpu.VMEM((IPW, B), jnp.float32),
            pltpu.VMEM((IPW, B), jnp.float32),
            pltpu.VMEM((QPW, B), jnp.float32),
            pltpu.SemaphoreType.DMA,
        ],
        compiler_params=pltpu.CompilerParams(use_tc_tiling_on_sc=False),
    )
    return run(zt, idxf, wexp)


def kernel(x, q, z):
    cv, ci = _candidates(x, q)
    cvt = cv.transpose(1, 0, 2).reshape(Q, C)
    cit = ci.transpose(1, 0, 2).reshape(Q, C)
    idx, w = _topk_weights(cvt, cit)
    zt = jnp.transpose(z)
    idxf = idx.reshape(Q * K)
    wexp = jnp.broadcast_to(w.reshape(Q * K, 1), (Q * K, B))
    out_t = _sc_gather_sum(zt, idxf, wexp)
    return jnp.transpose(out_t)


# speedup vs baseline: 1.1735x; 1.1735x over previous
import functools

import jax
import jax.numpy as jnp
from jax import lax
from jax.experimental import pallas as pl
from jax.experimental.pallas import tpu as pltpu
from jax.experimental.pallas import tpu_sc as plsc

K = 8
Q = 1024
D = 128
N = 100000
B = 16
QB = 1024
BLK = 2048
NSTEPS = -(-N // BLK)
NQB = Q // QB
C = NSTEPS * K

NC = 2
NS = 16
NW = NC * NS
QPW = Q // NW
IPW = QPW * K

INF = jnp.inf


def _cand_body(q_ref, x_ref, cv_out, ci_out, ubv, work_ref, fold_v, fold_i,
               fold2_v, fold2_i):
    j = pl.program_id(0)
    qb = pl.program_id(1)

    qv = q_ref[pl.ds(qb * QB, QB), :]
    xv = x_ref[...]
    qsq = jnp.sum(qv * qv, axis=1, keepdims=True)
    xsq = jnp.sum(xv * xv, axis=1)
    s = lax.dot_general(qv, xv, (((1,), (1,)), ((), ())),
                        preferred_element_type=jnp.float32)
    d2 = jnp.maximum(qsq + xsq[None, :] - 2.0 * s, 0.0)

    colsf = (jax.lax.broadcasted_iota(jnp.int32, (QB, BLK), 1)
             .astype(jnp.float32) + (j * BLK).astype(jnp.float32))

    qs = pl.ds(qb * QB, QB)

    @pl.when(j == 0)
    def _init():
        ubv[qs, :] = jnp.full((QB, K), INF, jnp.float32)

    thr = ubv[qs, K - 1:K]
    maskf = jnp.logical_and(d2 < thr, colsf < float(N)).astype(jnp.float32)
    cv_out[0, qs, :] = jnp.full((QB, K), INF, jnp.float32)
    ci_out[0, qs, :] = jnp.full((QB, K), INF, jnp.float32)

    P = jnp.where(maskf > 0.0, d2, INF)

    M = P[:, 0:128]
    Mi = jnp.zeros((QB, 128), jnp.float32)
    cnts = maskf[:, 0:128]
    for c in range(1, BLK // 128):
        v = P[:, c * 128:(c + 1) * 128]
        Mi = jnp.where(v < M, float(c), Mi)
        M = jnp.minimum(M, v)
        cnts = cnts + maskf[:, c * 128:(c + 1) * 128]
    cnt = jnp.max(jnp.sum(cnts, axis=1))
    ok = jnp.max(cnts) <= 1.0
    lanef = (jax.lax.broadcasted_iota(jnp.int32, (QB, 128), 1)
             .astype(jnp.float32))
    colglob = Mi * 128.0 + lanef + (j * BLK).astype(jnp.float32)

    @pl.when(ok)
    def _cheap():
        fold_v[...] = M
        fold_i[...] = colglob
        for k in range(K):
            @pl.when(cnt > float(k))
            def _p():
                w = fold_v[...]
                cg = fold_i[...]
                m = jnp.min(w, axis=1, keepdims=True)
                am = jnp.min(jnp.where(w == m, cg, INF), axis=1,
                             keepdims=True)
                cv_out[0, qs, K - 1 - k:K - k] = m
                ci_out[0, qs, K - 1 - k:K - k] = am
                if k < K - 1:
                    fold_v[...] = jnp.where(cg == am, INF, w)

    ok2 = jnp.max(cnts) <= 2.0

    @pl.when(jnp.logical_not(ok) & ok2)
    def _two():
        lo = P[:, 0:128]
        loi = jnp.zeros((QB, 128), jnp.float32)
        hi = jnp.full((QB, 128), INF, jnp.float32)
        hii = jnp.zeros((QB, 128), jnp.float32)
        for c in range(1, BLK // 128):
            t = P[:, c * 128:(c + 1) * 128]
            c1 = t < lo
            hii = jnp.where(c1, loi, jnp.where(t < hi, float(c), hii))
            hi = jnp.where(c1, lo, jnp.minimum(hi, t))
            loi = jnp.where(c1, float(c), loi)
            lo = jnp.minimum(lo, t)
        base = lanef + (j * BLK).astype(jnp.float32)
        fold2_v[:, 0:128] = lo
        fold2_v[:, 128:256] = hi
        fold2_i[:, 0:128] = loi * 128.0 + base
        fold2_i[:, 128:256] = hii * 128.0 + base
        for k in range(K):
            @pl.when(cnt > float(k))
            def _p2():
                w = fold2_v[...]
                cg = fold2_i[...]
                m = jnp.min(w, axis=1, keepdims=True)
                am = jnp.min(jnp.where(w == m, cg, INF), axis=1,
                             keepdims=True)
                cv_out[0, qs, K - 1 - k:K - k] = m
                ci_out[0, qs, K - 1 - k:K - k] = am
                if k < K - 1:
                    fold2_v[...] = jnp.where(cg == am, INF, w)

    @pl.when(jnp.logical_not(ok2))
    def _full():
        work_ref[...] = P
        for k in range(K):
            @pl.when(cnt > float(k))
            def _pass():
                work = work_ref[...]
                m = jnp.min(work, axis=1, keepdims=True)
                am = jnp.min(jnp.where(work == m, colsf, INF), axis=1,
                             keepdims=True)
                cv_out[0, qs, K - 1 - k:K - k] = m
                ci_out[0, qs, K - 1 - k:K - k] = am
                if k < K - 1:
                    work_ref[...] = jnp.where(colsf == am, INF, work)

    a = ubv[qs, :]
    b = cv_out[0, qs, :]
    L = jnp.minimum(a, b)
    lane = jax.lax.broadcasted_iota(jnp.int32, (QB, K), 1)
    for d in (4, 2, 1):
        fwd = jnp.concatenate(
            [L[:, d:], jnp.full((QB, d), INF, jnp.float32)], axis=1)
        bwd = jnp.concatenate(
            [jnp.full((QB, d), -INF, jnp.float32), L[:, :K - d]], axis=1)
        up = (lane // d) % 2 == 0
        L = jnp.where(up, jnp.minimum(L, fwd), jnp.maximum(L, bwd))
    ubv[qs, :] = L


def _candidates(x, q):
    return pl.pallas_call(
        _cand_body,
        grid=(NSTEPS, NQB),
        in_specs=[
            pl.BlockSpec((Q, D), lambda j, qb: (0, 0)),
            pl.BlockSpec((BLK, D), lambda j, qb: (j, 0)),
        ],
        out_specs=[
            pl.BlockSpec((1, Q, K), lambda j, qb: (j, 0, 0)),
            pl.BlockSpec((1, Q, K), lambda j, qb: (j, 0, 0)),
        ],
        out_shape=[
            jax.ShapeDtypeStruct((NSTEPS, Q, K), jnp.float32),
            jax.ShapeDtypeStruct((NSTEPS, Q, K), jnp.float32),
        ],
        scratch_shapes=[
            pltpu.VMEM((Q, K), jnp.float32),
            pltpu.VMEM((QB, BLK), jnp.float32),
            pltpu.VMEM((QB, 128), jnp.float32),
            pltpu.VMEM((QB, 128), jnp.float32),
            pltpu.VMEM((QB, 256), jnp.float32),
            pltpu.VMEM((QB, 256), jnp.float32),
        ],
    )(q, x)


def _final_body(cv_ref, ci_ref, idx_out, w_out):
    cw = cv_ref[...]
    ci = ci_ref[...]
    fv, fi = [], []
    for _ in range(K):
        m = jnp.min(cw, axis=1, keepdims=True)
        am = jnp.min(jnp.where(cw == m, ci, INF), axis=1, keepdims=True)
        fv.append(m)
        fi.append(am)
        cw = jnp.where(ci == am, INF, cw)
    d2t = jnp.concatenate(fv, axis=1)
    idx_out[...] = jnp.concatenate(fi, axis=1).astype(jnp.int32)
    dist = jnp.sqrt(d2t)
    sigma2 = jnp.square(jnp.max(dist)) / 9.0
    wgt = jnp.exp(-jnp.square(dist) / (2.0 * sigma2))
    w_out[...] = wgt / jnp.sum(wgt, axis=-1, keepdims=True)


def _topk_weights(cv, ci):
    return pl.pallas_call(
        _final_body,
        out_shape=[
            jax.ShapeDtypeStruct((Q, K), jnp.int32),
            jax.ShapeDtypeStruct((Q, K), jnp.float32),
        ],
    )(cv, ci)


def _sc_body(zt_hbm, idx_hbm, w_hbm, out_hbm, idx_v, w_v, rows_v, acc_v, sem):
    c = lax.axis_index("c")
    s = lax.axis_index("s")
    wid = s * NC + c
    base = wid * IPW
    qbase = wid * QPW
    pltpu.sync_copy(idx_hbm.at[pl.ds(base, IPW)], idx_v)
    pltpu.sync_copy(w_hbm.at[pl.ds(base, IPW)], w_v)
    pltpu.async_copy(zt_hbm.at[idx_v], rows_v, sem).wait()
    for i in range(QPW):
        acc = rows_v[i * K, :] * w_v[i * K, :]
        for k in range(1, K):
            acc = acc + rows_v[i * K + k, :] * w_v[i * K + k, :]
        acc_v[i, :] = acc
    pltpu.sync_copy(acc_v, out_hbm.at[pl.ds(qbase, QPW)])


def _sc_gather_sum(zt, idxf, wexp):
    run = pl.kernel(
        _sc_body,
        out_type=jax.ShapeDtypeStruct((Q, B), jnp.float32),
        mesh=plsc.VectorSubcoreMesh(core_axis_name="c", subcore_axis_name="s",
                                    num_cores=NC, num_subcores=NS),
        scratch_types=[
            pltpu.VMEM((IPW,), jnp.int32),
            pltpu.VMEM((IPW, B), jnp.float32),
            pltpu.VMEM((IPW, B), jnp.float32),
            pltpu.VMEM((QPW, B), jnp.float32),
            pltpu.SemaphoreType.DMA,
        ],
        compiler_params=pltpu.CompilerParams(use_tc_tiling_on_sc=False),
    )
    return run(zt, idxf, wexp)


def kernel(x, q, z):
    cv, ci = _candidates(x, q)
    cvt = cv.transpose(1, 0, 2).reshape(Q, C)
    cit = ci.transpose(1, 0, 2).reshape(Q, C)
    idx, w = _topk_weights(cvt, cit)
    zt = jnp.transpose(z)
    idxf = idx.reshape(Q * K)
    wexp = jnp.broadcast_to(w.reshape(Q * K, 1), (Q * K, B))
    out_t = _sc_gather_sum(zt, idxf, wexp)
    return jnp.transpose(out_t)
